# baseline (speedup 1.0000x reference)
import functools

import jax
import jax.numpy as jnp
from jax import lax
from jax.experimental import pallas as pl
from jax.experimental.pallas import tpu as pltpu
from jax.experimental.pallas import tpu_sc as plsc

_T = 8192
_N = 2048
_NU = 1536

_NC = 2
_NS = 16
_NW = _NC * _NS
_R = 4
_ROWS_PER_W = _T // _NW
_CHUNKS = _ROWS_PER_W // _R
_NJV = _NU // 16
_NOBS = 8
_NUN = 4
_D = 4

_mesh = plsc.VectorSubcoreMesh(core_axis_name="c", subcore_axis_name="s")


@functools.partial(
    pl.kernel,
    mesh=_mesh,
    out_type=jax.ShapeDtypeStruct((_T, _N), jnp.float32),
    compiler_params=pltpu.CompilerParams(needs_layout_passes=False),
    scratch_types=(
        [pltpu.VMEM((_R, _N), jnp.float32) for _ in range(_NOBS)]
        + [pltpu.VMEM((_R, _NU), jnp.float32) for _ in range(_NUN)]
        + [pltpu.VMEM((_NU,), jnp.int32)]
        + [pltpu.SemaphoreType.DMA for _ in range(_NOBS + _NUN + _NOBS)]
    ),
)
def _assemble(uobs_hbm, uunobs_hbm, cidx_hbm, out_hbm,
              obs0, obs1, obs2, obs3, obs4, obs5, obs6, obs7,
              un0, un1, un2, un3, cidx_v,
              iob0, iob1, iob2, iob3, iob4, iob5, iob6, iob7,
              iun0, iun1, iun2, iun3,
              osem0, osem1, osem2, osem3, osem4, osem5, osem6, osem7):
    obs = (obs0, obs1, obs2, obs3, obs4, obs5, obs6, obs7)
    un = (un0, un1, un2, un3)
    iob = (iob0, iob1, iob2, iob3, iob4, iob5, iob6, iob7)
    iun = (iun0, iun1, iun2, iun3)
    osem = (osem0, osem1, osem2, osem3, osem4, osem5, osem6, osem7)

    wid = lax.axis_index("s") * _NC + lax.axis_index("c")
    row0 = wid * _ROWS_PER_W

    def start_in(g, jo, ju):
        base = row0 + g * _R
        pltpu.async_copy(uobs_hbm.at[pl.ds(base, _R), :], obs[jo], iob[jo])
        pltpu.async_copy(uunobs_hbm.at[pl.ds(base, _R), :], un[ju], iun[ju])

    def wait_in(g, jo, ju):
        base = row0 + g * _R
        pltpu.make_async_copy(uobs_hbm.at[pl.ds(base, _R), :],
                              obs[jo], iob[jo]).wait()
        pltpu.make_async_copy(uunobs_hbm.at[pl.ds(base, _R), :],
                              un[ju], iun[ju]).wait()

    def start_out(g, jo):
        base = row0 + g * _R
        pltpu.async_copy(obs[jo], out_hbm.at[pl.ds(base, _R), :], osem[jo])

    def wait_out(g, jo):
        base = row0 + g * _R
        pltpu.make_async_copy(obs[jo], out_hbm.at[pl.ds(base, _R), :],
                              osem[jo]).wait()

    for g0 in range(_D):
        start_in(g0, g0, g0 % _NUN)
    pltpu.sync_copy(cidx_hbm, cidx_v)

    rivs = [jnp.full((16,), r, jnp.int32) for r in range(_R)]

    def scatter(jo, ju):
        obs_v = obs[jo]
        un_v = un[ju]

        @plsc.parallel_loop(0, _NJV, unroll=4)
        def body(j):
            off = j * 16
            civ = cidx_v[pl.ds(off, 16)]
            for r in range(_R):
                vv = un_v[r, pl.ds(off, 16)]
                plsc.addupdate_scatter(obs_v, [rivs[r], civ], vv)

    def outer(k, carry):
        for j in range(_NOBS):
            g = _NOBS * k + j
            jo = j
            ju = j % _NUN
            wait_in(g, jo, ju)
            scatter(jo, ju)
            start_out(g, jo)
            jp = (j + _D) % _NOBS
            if j < _D:
                @pl.when(k >= 1)
                def _():
                    wait_out(g - _D, jp)
                start_in(g + _D, jp, ju)
            else:
                @pl.when(k < (_CHUNKS // _NOBS) - 1)
                def _():
                    wait_out(g - _D, jp)
                    start_in(g + _D, jp, ju)
        return carry

    lax.fori_loop(0, _CHUNKS // _NOBS, outer, 0)

    for g in range(_CHUNKS - _NOBS, _CHUNKS):
        wait_out(g, g % _NOBS)


def kernel(u_obs, u_unobs, unobs_idx):
    return _assemble(u_obs, u_unobs, unobs_idx)


# speedup vs baseline: 1.0095x; 1.0095x over previous
import functools

import jax
import jax.numpy as jnp
from jax import lax
from jax.experimental import pallas as pl
from jax.experimental.pallas import tpu as pltpu
from jax.experimental.pallas import tpu_sc as plsc

_T = 8192
_N = 2048
_NU = 1536

_NC = 2
_NS = 16
_NW = _NC * _NS
_R = 4
_ROWS_PER_W = _T // _NW
_CHUNKS = _ROWS_PER_W // _R
_NJV = _NU // 16
_NOBS = 8
_NUN = 4
_D = 4

_mesh = plsc.VectorSubcoreMesh(core_axis_name="c", subcore_axis_name="s")


@functools.partial(
    pl.kernel,
    mesh=_mesh,
    out_type=jax.ShapeDtypeStruct((_T, _N), jnp.float32),
    compiler_params=pltpu.CompilerParams(needs_layout_passes=False),
    scratch_types=(
        [pltpu.VMEM((_R, _N), jnp.float32) for _ in range(_NOBS)]
        + [pltpu.VMEM((_R, _NU), jnp.float32) for _ in range(_NUN)]
        + [pltpu.VMEM((_NU,), jnp.int32)]
        + [pltpu.SemaphoreType.DMA for _ in range(_NOBS + _NUN + _NOBS)]
    ),
)
def _assemble(uobs_hbm, uunobs_hbm, cidx_hbm, out_hbm,
              obs0, obs1, obs2, obs3, obs4, obs5, obs6, obs7,
              un0, un1, un2, un3, cidx_v,
              iob0, iob1, iob2, iob3, iob4, iob5, iob6, iob7,
              iun0, iun1, iun2, iun3,
              osem0, osem1, osem2, osem3, osem4, osem5, osem6, osem7):
    obs = (obs0, obs1, obs2, obs3, obs4, obs5, obs6, obs7)
    un = (un0, un1, un2, un3)
    iob = (iob0, iob1, iob2, iob3, iob4, iob5, iob6, iob7)
    iun = (iun0, iun1, iun2, iun3)
    osem = (osem0, osem1, osem2, osem3, osem4, osem5, osem6, osem7)

    wid = lax.axis_index("s") * _NC + lax.axis_index("c")
    row0 = wid * _ROWS_PER_W

    def start_in(g, jo, ju):
        base = row0 + g * _R
        pltpu.async_copy(uobs_hbm.at[pl.ds(base, _R), :], obs[jo], iob[jo])
        pltpu.async_copy(uunobs_hbm.at[pl.ds(base, _R), :], un[ju], iun[ju])

    def wait_in(g, jo, ju):
        base = row0 + g * _R
        pltpu.make_async_copy(uobs_hbm.at[pl.ds(base, _R), :],
                              obs[jo], iob[jo]).wait()
        pltpu.make_async_copy(uunobs_hbm.at[pl.ds(base, _R), :],
                              un[ju], iun[ju]).wait()

    def start_out(g, jo):
        base = row0 + g * _R
        pltpu.async_copy(obs[jo], out_hbm.at[pl.ds(base, _R), :], osem[jo])

    def wait_out(g, jo):
        base = row0 + g * _R
        pltpu.make_async_copy(obs[jo], out_hbm.at[pl.ds(base, _R), :],
                              osem[jo]).wait()

    for g0 in range(_D):
        start_in(g0, g0, g0 % _NUN)
    pltpu.sync_copy(cidx_hbm, cidx_v)

    rivs = [jnp.full((16,), r, jnp.int32) for r in range(_R)]

    def scatter(jo, ju):
        obs_v = obs[jo]
        un_v = un[ju]

        @plsc.parallel_loop(0, _NJV, unroll=1)
        def body(j):
            off = j * 16
            civ = cidx_v[pl.ds(off, 16)]
            for r in range(_R):
                vv = un_v[r, pl.ds(off, 16)]
                plsc.addupdate_scatter(obs_v, [rivs[r], civ], vv)

    def outer(k, carry):
        for j in range(_NOBS):
            g = _NOBS * k + j
            jo = j
            ju = j % _NUN
            wait_in(g, jo, ju)
            scatter(jo, ju)
            start_out(g, jo)
            jp = (j + _D) % _NOBS
            if j < _D:
                @pl.when(k >= 1)
                def _():
                    wait_out(g - _D, jp)
                start_in(g + _D, jp, ju)
            else:
                @pl.when(k < (_CHUNKS // _NOBS) - 1)
                def _():
                    wait_out(g - _D, jp)
                    start_in(g + _D, jp, ju)
        return carry

    lax.fori_loop(0, _CHUNKS // _NOBS, outer, 0)

    for g in range(_CHUNKS - _NOBS, _CHUNKS):
        wait_out(g, g % _NOBS)


def kernel(u_obs, u_unobs, unobs_idx):
    return _assemble(u_obs, u_unobs, unobs_idx)
